# baseline (speedup 1.0000x reference)
import functools

import jax
import jax.numpy as jnp
from jax.experimental import pallas as pl
from jax.experimental.pallas import tpu as pltpu

_B = 64
_D = 64
_M = 65536
_TILE = 4096
_T = _M // _TILE


def _body(x_ref, k_ref, v_ref, w_ref, p_ref, w_scr, sum_scr, acc_scr):
    p = pl.program_id(0)
    t = pl.program_id(1)

    @pl.when(jnp.logical_and(p == 0, t == 0))
    def _init():
        sum_scr[...] = jnp.zeros_like(sum_scr)
        acc_scr[...] = jnp.zeros_like(acc_scr)

    @pl.when(p == 0)
    def _compute():
        xv = x_ref[...]
        kv = k_ref[...]
        num = jax.lax.dot_general(
            xv, kv, (((1,), (1,)), ((), ())),
            preferred_element_type=jnp.float32)
        c = 10.0 * 1.4426950408889634
        inv_xn = c * jax.lax.rsqrt(
            jnp.sum(xv * xv, axis=1, keepdims=True))
        k2 = jax.lax.dot_general(
            jnp.ones((1, _D), jnp.float32), kv * kv,
            (((1,), (1,)), ((), ())),
            preferred_element_type=jnp.float32,
            precision=jax.lax.Precision.HIGHEST)
        inv_kn = jax.lax.rsqrt(k2)
        scale = jnp.minimum(inv_xn * inv_kn, c * 1e8)
        e = jnp.exp2(num * scale)
        w_scr[t] = e
        sum_scr[...] += jax.lax.dot_general(
            e, jnp.ones((_TILE, 1), jnp.float32), (((1,), (0,)), ((), ())),
            preferred_element_type=jnp.float32)
        acc_scr[...] += jnp.dot(e, v_ref[...],
                                preferred_element_type=jnp.float32)

    @pl.when(p == 1)
    def _normalize():
        inv = 1.0 / sum_scr[...]
        w_ref[...] = w_scr[t] * inv
        p_ref[...] = acc_scr[...] * inv


@jax.jit
def kernel(x, memory_keys, memory_values):
    weights, pred = pl.pallas_call(
        _body,
        grid=(2, _T),
        in_specs=[
            pl.BlockSpec((_B, _D), lambda p, t: (0, 0)),
            pl.BlockSpec((_TILE, _D), lambda p, t: (t * (1 - p), 0)),
            pl.BlockSpec((_TILE, _D), lambda p, t: (t * (1 - p), 0)),
        ],
        out_specs=[
            pl.BlockSpec((_B, _TILE), lambda p, t: (0, t * p)),
            pl.BlockSpec((_B, _D), lambda p, t: (0, 0)),
        ],
        out_shape=[
            jax.ShapeDtypeStruct((_B, _M), jnp.float32),
            jax.ShapeDtypeStruct((_B, _D), jnp.float32),
        ],
        scratch_shapes=[
            pltpu.VMEM((_T, _B, _TILE), jnp.float32),
            pltpu.VMEM((_B, 1), jnp.float32),
            pltpu.VMEM((_B, _D), jnp.float32),
        ],
    )(x, memory_keys, memory_values)
    return (pred, weights)


# speedup vs baseline: 1.1532x; 1.1532x over previous
import functools

import jax
import jax.numpy as jnp
from jax.experimental import pallas as pl
from jax.experimental.pallas import tpu as pltpu

_B = 64
_D = 64
_M = 65536
_TILE = 4096
_T = _M // _TILE


def _body(x_ref, k_ref, v_ref, w_ref, p_ref, w_scr, sum_scr, acc_scr):
    p = pl.program_id(0)
    t = pl.program_id(1)

    @pl.when(jnp.logical_and(p == 0, t == 0))
    def _init():
        sum_scr[...] = jnp.zeros_like(sum_scr)
        acc_scr[...] = jnp.zeros_like(acc_scr)

    @pl.when(p == 0)
    def _compute():
        xv = x_ref[...]
        kb = k_ref[...].astype(jnp.bfloat16)
        vb = v_ref[...].astype(jnp.bfloat16)
        num = jax.lax.dot_general(
            xv.astype(jnp.bfloat16), kb, (((1,), (1,)), ((), ())),
            preferred_element_type=jnp.float32)
        c = 10.0 * 1.4426950408889634
        inv_xn = c * jax.lax.rsqrt(
            jnp.sum(xv * xv, axis=1, keepdims=True))
        k2 = jax.lax.dot_general(
            jnp.ones((1, _D), jnp.bfloat16), kb * kb,
            (((1,), (1,)), ((), ())),
            preferred_element_type=jnp.float32)
        inv_kn = jax.lax.rsqrt(k2)
        scale = jnp.minimum(inv_xn * inv_kn, c * 1e8)
        e = jnp.exp2(num * scale)
        w_scr[t] = e
        eb = e.astype(jnp.bfloat16)
        sum_scr[...] += jax.lax.dot_general(
            eb, jnp.ones((_TILE, 1), jnp.bfloat16), (((1,), (0,)), ((), ())),
            preferred_element_type=jnp.float32)
        acc_scr[...] += jnp.dot(eb, vb, preferred_element_type=jnp.float32)

    @pl.when(p == 1)
    def _normalize():
        inv = 1.0 / sum_scr[...]
        w_ref[...] = w_scr[t] * inv
        p_ref[...] = acc_scr[...] * inv


@jax.jit
def kernel(x, memory_keys, memory_values):
    weights, pred = pl.pallas_call(
        _body,
        grid=(2, _T),
        in_specs=[
            pl.BlockSpec((_B, _D), lambda p, t: (0, 0)),
            pl.BlockSpec((_TILE, _D), lambda p, t: (t * (1 - p), 0)),
            pl.BlockSpec((_TILE, _D), lambda p, t: (t * (1 - p), 0)),
        ],
        out_specs=[
            pl.BlockSpec((_B, _TILE), lambda p, t: (0, t * p)),
            pl.BlockSpec((_B, _D), lambda p, t: (0, 0)),
        ],
        out_shape=[
            jax.ShapeDtypeStruct((_B, _M), jnp.float32),
            jax.ShapeDtypeStruct((_B, _D), jnp.float32),
        ],
        scratch_shapes=[
            pltpu.VMEM((_T, _B, _TILE), jnp.float32),
            pltpu.VMEM((_B, 1), jnp.float32),
            pltpu.VMEM((_B, _D), jnp.float32),
        ],
    )(x, memory_keys, memory_values)
    return (pred, weights)


# speedup vs baseline: 1.2221x; 1.0597x over previous
import functools

import jax
import jax.numpy as jnp
from jax.experimental import pallas as pl
from jax.experimental.pallas import tpu as pltpu

_B = 64
_D = 64
_M = 65536
_TILE = 8192
_T = _M // _TILE


def _body(x_ref, k_ref, v_ref, w_ref, p_ref, w_scr, sum_scr, acc_scr):
    p = pl.program_id(0)
    t = pl.program_id(1)

    @pl.when(jnp.logical_and(p == 0, t == 0))
    def _init():
        sum_scr[...] = jnp.zeros_like(sum_scr)
        acc_scr[...] = jnp.zeros_like(acc_scr)

    @pl.when(p == 0)
    def _compute():
        xv = x_ref[...]
        kb = k_ref[...].astype(jnp.bfloat16)
        vb = v_ref[...].astype(jnp.bfloat16)
        num = jax.lax.dot_general(
            xv.astype(jnp.bfloat16), kb, (((1,), (1,)), ((), ())),
            preferred_element_type=jnp.float32)
        c = 10.0 * 1.4426950408889634
        inv_xn = c * jax.lax.rsqrt(
            jnp.sum(xv * xv, axis=1, keepdims=True))
        k2 = jax.lax.dot_general(
            jnp.ones((1, _D), jnp.bfloat16), kb * kb,
            (((1,), (1,)), ((), ())),
            preferred_element_type=jnp.float32)
        inv_kn = jax.lax.rsqrt(k2)
        scale = jnp.minimum(inv_xn * inv_kn, c * 1e8)
        e = jnp.exp2(num * scale)
        w_scr[t] = e
        eb = e.astype(jnp.bfloat16)
        sum_scr[...] += jax.lax.dot_general(
            eb, jnp.ones((_TILE, 1), jnp.bfloat16), (((1,), (0,)), ((), ())),
            preferred_element_type=jnp.float32)
        acc_scr[...] += jnp.dot(eb, vb, preferred_element_type=jnp.float32)

    @pl.when(p == 1)
    def _normalize():
        inv = 1.0 / sum_scr[...]
        w_ref[...] = w_scr[t] * inv
        p_ref[...] = acc_scr[...] * inv


@jax.jit
def kernel(x, memory_keys, memory_values):
    weights, pred = pl.pallas_call(
        _body,
        grid=(2, _T),
        in_specs=[
            pl.BlockSpec((_B, _D), lambda p, t: (0, 0)),
            pl.BlockSpec((_TILE, _D), lambda p, t: (t * (1 - p), 0)),
            pl.BlockSpec((_TILE, _D), lambda p, t: (t * (1 - p), 0)),
        ],
        out_specs=[
            pl.BlockSpec((_B, _TILE), lambda p, t: (0, t * p)),
            pl.BlockSpec((_B, _D), lambda p, t: (0, 0)),
        ],
        out_shape=[
            jax.ShapeDtypeStruct((_B, _M), jnp.float32),
            jax.ShapeDtypeStruct((_B, _D), jnp.float32),
        ],
        scratch_shapes=[
            pltpu.VMEM((_T, _B, _TILE), jnp.float32),
            pltpu.VMEM((_B, 1), jnp.float32),
            pltpu.VMEM((_B, _D), jnp.float32),
        ],
    )(x, memory_keys, memory_values)
    return (pred, weights)
